# baseline (speedup 1.0000x reference)
import functools

import jax
import jax.numpy as jnp
from jax import lax
from jax.experimental import pallas as pl
from jax.experimental.pallas import tpu as pltpu
from jax.experimental.pallas import tpu_sc as plsc

DIM = 4096
E = 64
K = 8
T_BLOCK = 1024
NEG_INF = float("-inf")

NC = 2
NS = 16
NW = NC * NS
LANES = 16


def _noisy_logits_block(x_ref, w_ref, noise_ref, out_ref):
    logits = jnp.dot(x_ref[...], w_ref[...], preferred_element_type=jnp.float32)
    std = jax.nn.softplus(logits[:, E:]) + 1e-2
    out_ref[...] = (logits[:, :E] + noise_ref[...] * std).T


def _tc_noisy_logits_t(x, noise, w):
    t = x.shape[0]
    return pl.pallas_call(
        _noisy_logits_block,
        grid=(t // T_BLOCK,),
        in_specs=[
            pl.BlockSpec((T_BLOCK, DIM), lambda i: (i, 0)),
            pl.BlockSpec((DIM, 2 * E), lambda i: (0, 0)),
            pl.BlockSpec((T_BLOCK, E), lambda i: (i, 0)),
        ],
        out_specs=pl.BlockSpec((E, T_BLOCK), lambda i: (0, i)),
        out_shape=jax.ShapeDtypeStruct((E, t), jnp.float32),
    )(x, w, noise)


def _vmax_tree(ws):
    while len(ws) > 1:
        nxt = [jnp.maximum(a, b) for a, b in zip(ws[0::2], ws[1::2])]
        if len(ws) % 2:
            nxt.append(ws[-1])
        ws = nxt
    return ws[0]


def _vadd_tree(ws):
    while len(ws) > 1:
        nxt = [a + b for a, b in zip(ws[0::2], ws[1::2])]
        if len(ws) % 2:
            nxt.append(ws[-1])
        ws = nxt
    return ws[0]


def _sc_route_body(noisy_hbm, out_hbm, buf_in, buf_out):
    rows = buf_in.shape[1]
    wid = lax.axis_index("s") * NC + lax.axis_index("c")
    base = wid * rows
    pltpu.sync_copy(noisy_hbm.at[:, pl.ds(base, rows)], buf_in)

    @plsc.parallel_loop(0, rows // LANES, 1, unroll=4)
    def group(g):
        sl = pl.ds(g * LANES, LANES)
        vs = [buf_in[j, sl] for j in range(E)]
        m = [jnp.full((LANES,), NEG_INF, jnp.float32)] * K
        for v in vs:
            c = v
            for i in range(K):
                hi = jnp.maximum(m[i], c)
                c = jnp.minimum(m[i], c)
                m[i] = hi
        rowmax, thresh = m[0], m[K - 1]
        es = [jnp.where(v >= thresh, jnp.exp(v - rowmax), 0.0) for v in vs]
        inv = 1.0 / _vadd_tree(es)
        for j in range(E):
            buf_out[j, sl] = es[j] * inv

    pltpu.sync_copy(buf_out, out_hbm.at[:, pl.ds(base, rows)])


def _sc_route(noisy_t):
    t = noisy_t.shape[1]
    rows = t // NW
    mesh = plsc.VectorSubcoreMesh(core_axis_name="c", subcore_axis_name="s")
    return pl.kernel(
        _sc_route_body,
        out_type=jax.ShapeDtypeStruct((E, t), jnp.float32),
        mesh=mesh,
        scratch_types=[
            pltpu.VMEM((E, rows), jnp.float32),
            pltpu.VMEM((E, rows), jnp.float32),
        ],
    )(noisy_t)


@jax.jit
def kernel(x, noise, Wg, Wnoise):
    t = x.shape[0]
    w = jnp.concatenate([Wg, Wnoise], axis=1)
    noisy_t = _tc_noisy_logits_t(x, noise, w)
    return _sc_route(noisy_t).T


# speedup vs baseline: 1.1215x; 1.1215x over previous
import functools

import jax
import jax.numpy as jnp
from jax import lax
from jax.experimental import pallas as pl
from jax.experimental.pallas import tpu as pltpu
from jax.experimental.pallas import tpu_sc as plsc

DIM = 4096
E = 64
K = 8
T_BLOCK = 1024
NEG_INF = float("-inf")

NC = 2
NS = 16
NW = NC * NS
LANES = 16


def _noisy_logits_block(x_ref, w_ref, noise_ref, out_ref):
    logits = jnp.dot(x_ref[...], w_ref[...], preferred_element_type=jnp.float32)
    std = jax.nn.softplus(logits[:, E:]) + 1e-2
    out_ref[...] = (logits[:, :E] + noise_ref[...] * std).T


def _tc_noisy_logits_t(x, noise, w):
    t = x.shape[0]
    return pl.pallas_call(
        _noisy_logits_block,
        grid=(t // T_BLOCK,),
        in_specs=[
            pl.BlockSpec((T_BLOCK, DIM), lambda i: (i, 0)),
            pl.BlockSpec((DIM, 2 * E), lambda i: (0, 0)),
            pl.BlockSpec((T_BLOCK, E), lambda i: (i, 0)),
        ],
        out_specs=pl.BlockSpec((E, T_BLOCK), lambda i: (0, i)),
        out_shape=jax.ShapeDtypeStruct((E, t), jnp.float32),
    )(x, w, noise)


def _vmax_tree(ws):
    while len(ws) > 1:
        nxt = [jnp.maximum(a, b) for a, b in zip(ws[0::2], ws[1::2])]
        if len(ws) % 2:
            nxt.append(ws[-1])
        ws = nxt
    return ws[0]


def _vadd_tree(ws):
    while len(ws) > 1:
        nxt = [a + b for a, b in zip(ws[0::2], ws[1::2])]
        if len(ws) % 2:
            nxt.append(ws[-1])
        ws = nxt
    return ws[0]


def _route_groups(buf_in, buf_out, ngroups):
    @plsc.parallel_loop(0, ngroups, 1, unroll=2)
    def group(g):
        sl = pl.ds(g * LANES, LANES)
        vs = [buf_in[j, sl] for j in range(E)]
        m = [jnp.full((LANES,), NEG_INF, jnp.float32)] * K
        for v in vs:
            c = v
            for i in range(K):
                hi = jnp.maximum(m[i], c)
                c = jnp.minimum(m[i], c)
                m[i] = hi
        rowmax, thresh = m[0], m[K - 1]
        es = [jnp.where(v >= thresh, jnp.exp(v - rowmax), 0.0) for v in vs]
        inv = 1.0 / _vadd_tree(es)
        for j in range(E):
            buf_out[j, sl] = es[j] * inv


def _sc_route_body(noisy_hbm, out_hbm,
                   bin0, bin1, bout0, bout1, sin0, sin1, sout0, sout1):
    half = bin0.shape[1]
    wid = lax.axis_index("s") * NC + lax.axis_index("c")
    base = wid * (2 * half)
    cin0 = pltpu.make_async_copy(noisy_hbm.at[:, pl.ds(base, half)], bin0, sin0)
    cin1 = pltpu.make_async_copy(
        noisy_hbm.at[:, pl.ds(base + half, half)], bin1, sin1)
    cin0.start()
    cin1.start()
    cin0.wait()
    _route_groups(bin0, bout0, half // LANES)
    cout0 = pltpu.make_async_copy(bout0, out_hbm.at[:, pl.ds(base, half)], sout0)
    cout0.start()
    cin1.wait()
    _route_groups(bin1, bout1, half // LANES)
    cout1 = pltpu.make_async_copy(
        bout1, out_hbm.at[:, pl.ds(base + half, half)], sout1)
    cout1.start()
    cout0.wait()
    cout1.wait()


def _sc_route(noisy_t):
    t = noisy_t.shape[1]
    rows = t // NW
    mesh = plsc.VectorSubcoreMesh(core_axis_name="c", subcore_axis_name="s")
    return pl.kernel(
        _sc_route_body,
        out_type=jax.ShapeDtypeStruct((E, t), jnp.float32),
        mesh=mesh,
        scratch_types=[
            pltpu.VMEM((E, rows // 2), jnp.float32),
            pltpu.VMEM((E, rows // 2), jnp.float32),
            pltpu.VMEM((E, rows // 2), jnp.float32),
            pltpu.VMEM((E, rows // 2), jnp.float32),
            pltpu.SemaphoreType.DMA,
            pltpu.SemaphoreType.DMA,
            pltpu.SemaphoreType.DMA,
            pltpu.SemaphoreType.DMA,
        ],
    )(noisy_t)


@jax.jit
def kernel(x, noise, Wg, Wnoise):
    t = x.shape[0]
    w = jnp.concatenate([Wg, Wnoise], axis=1)
    noisy_t = _tc_noisy_logits_t(x, noise, w)
    return _sc_route(noisy_t).T


# speedup vs baseline: 1.1297x; 1.0074x over previous
import functools

import jax
import jax.numpy as jnp
from jax import lax
from jax.experimental import pallas as pl
from jax.experimental.pallas import tpu as pltpu
from jax.experimental.pallas import tpu_sc as plsc

DIM = 4096
E = 64
K = 8
T_BLOCK = 1024
NEG_INF = float("-inf")

NC = 2
NS = 16
NW = NC * NS
LANES = 16


def _noisy_logits_block(x_ref, w_ref, noise_ref, out_ref):
    logits = jnp.dot(x_ref[...], w_ref[...], preferred_element_type=jnp.float32)
    std = jax.nn.softplus(logits[:, E:]) + 1e-2
    out_ref[...] = (logits[:, :E] + noise_ref[...] * std).T


def _tc_noisy_logits_t(x, noise, w):
    t = x.shape[0]
    return pl.pallas_call(
        _noisy_logits_block,
        grid=(t // T_BLOCK,),
        in_specs=[
            pl.BlockSpec((T_BLOCK, DIM), lambda i: (i, 0)),
            pl.BlockSpec((DIM, 2 * E), lambda i: (0, 0)),
            pl.BlockSpec((T_BLOCK, E), lambda i: (i, 0)),
        ],
        out_specs=pl.BlockSpec((E, T_BLOCK), lambda i: (0, i)),
        out_shape=jax.ShapeDtypeStruct((E, t), jnp.float32),
    )(x, w, noise)


def _vmax_tree(ws):
    while len(ws) > 1:
        nxt = [jnp.maximum(a, b) for a, b in zip(ws[0::2], ws[1::2])]
        if len(ws) % 2:
            nxt.append(ws[-1])
        ws = nxt
    return ws[0]


def _vadd_tree(ws):
    while len(ws) > 1:
        nxt = [a + b for a, b in zip(ws[0::2], ws[1::2])]
        if len(ws) % 2:
            nxt.append(ws[-1])
        ws = nxt
    return ws[0]


def _route_groups(buf_in, buf_out, ngroups):
    @plsc.parallel_loop(0, ngroups, 1, unroll=2)
    def group(g):
        sl = pl.ds(g * LANES, LANES)
        vs = [buf_in[j, sl] for j in range(E)]
        m = [jnp.full((LANES,), NEG_INF, jnp.float32)] * K
        for v in vs:
            c = v
            for i in range(K):
                hi = jnp.maximum(m[i], c)
                c = jnp.minimum(m[i], c)
                m[i] = hi
        rowmax, thresh = m[0], m[K - 1]
        es = [jnp.where(v >= thresh, jnp.exp(v - rowmax), 0.0) for v in vs]
        inv = 1.0 / _vadd_tree(es)
        for j in range(E):
            buf_out[j, sl] = es[j] * inv


def _sc_route_body(noisy_hbm, out_hbm, buf_in, buf_out):
    rows = buf_in.shape[1]
    wid = lax.axis_index("s") * NC + lax.axis_index("c")
    base = wid * rows
    pltpu.sync_copy(noisy_hbm.at[:, pl.ds(base, rows)], buf_in)
    _route_groups(buf_in, buf_out, rows // LANES)
    pltpu.sync_copy(buf_out, out_hbm.at[:, pl.ds(base, rows)])


def _sc_route(noisy_t):
    t = noisy_t.shape[1]
    rows = t // NW
    mesh = plsc.VectorSubcoreMesh(core_axis_name="c", subcore_axis_name="s")
    return pl.kernel(
        _sc_route_body,
        out_type=jax.ShapeDtypeStruct((E, t), jnp.float32),
        mesh=mesh,
        scratch_types=[
            pltpu.VMEM((E, rows), jnp.float32),
            pltpu.VMEM((E, rows), jnp.float32),
        ],
    )(noisy_t)


@jax.jit
def kernel(x, noise, Wg, Wnoise):
    t = x.shape[0]
    w = jnp.concatenate([Wg, Wnoise], axis=1)
    noisy_t = _tc_noisy_logits_t(x, noise, w)
    return _sc_route(noisy_t).T


# speedup vs baseline: 1.1487x; 1.0168x over previous
import functools

import jax
import jax.numpy as jnp
from jax import lax
from jax.experimental import pallas as pl
from jax.experimental.pallas import tpu as pltpu
from jax.experimental.pallas import tpu_sc as plsc

DIM = 4096
E = 64
K = 8
T_BLOCK = 1024
NEG_INF = float("-inf")

NC = 2
NS = 16
NW = NC * NS
LANES = 16


def _noisy_logits_block(x_ref, w_ref, noise_ref, out_ref):
    logits = jnp.dot(x_ref[...], w_ref[...], preferred_element_type=jnp.float32)
    std = jax.nn.softplus(logits[:, E:]) + 1e-2
    out_ref[...] = (logits[:, :E] + noise_ref[...] * std).T


def _tc_noisy_logits_t(x, noise, w):
    t = x.shape[0]
    return pl.pallas_call(
        _noisy_logits_block,
        grid=(t // T_BLOCK,),
        in_specs=[
            pl.BlockSpec((T_BLOCK, DIM), lambda i: (i, 0)),
            pl.BlockSpec((DIM, 2 * E), lambda i: (0, 0)),
            pl.BlockSpec((T_BLOCK, E), lambda i: (i, 0)),
        ],
        out_specs=pl.BlockSpec((E, T_BLOCK), lambda i: (0, i)),
        out_shape=jax.ShapeDtypeStruct((E, t), jnp.float32),
    )(x, w, noise)


def _vmax_tree(ws):
    while len(ws) > 1:
        nxt = [jnp.maximum(a, b) for a, b in zip(ws[0::2], ws[1::2])]
        if len(ws) % 2:
            nxt.append(ws[-1])
        ws = nxt
    return ws[0]


def _vadd_tree(ws):
    while len(ws) > 1:
        nxt = [a + b for a, b in zip(ws[0::2], ws[1::2])]
        if len(ws) % 2:
            nxt.append(ws[-1])
        ws = nxt
    return ws[0]


_SORT8 = ((0, 1), (2, 3), (4, 5), (6, 7),
          (0, 2), (1, 3), (4, 6), (5, 7),
          (1, 2), (5, 6),
          (0, 4), (1, 5), (2, 6), (3, 7),
          (2, 4), (3, 5),
          (1, 2), (3, 4), (5, 6))


def _sort8_desc(v):
    v = list(v)
    for i, j in _SORT8:
        hi = jnp.maximum(v[i], v[j])
        v[j] = jnp.minimum(v[i], v[j])
        v[i] = hi
    return v


def _merge_top8_desc(a, b):
    c = [jnp.maximum(a[i], b[K - 1 - i]) for i in range(K)]
    for d in (4, 2, 1):
        for i in range(K):
            if (i & d) == 0 and i + d < K:
                hi = jnp.maximum(c[i], c[i + d])
                c[i + d] = jnp.minimum(c[i], c[i + d])
                c[i] = hi
    return c


def _route_groups(buf_in, buf_out, ngroups):
    @plsc.parallel_loop(0, ngroups, 1, unroll=2)
    def group(g):
        sl = pl.ds(g * LANES, LANES)
        vs = [buf_in[j, sl] for j in range(E)]
        lists = [_sort8_desc(vs[K * b:K * b + K]) for b in range(E // K)]
        while len(lists) > 2:
            lists = [_merge_top8_desc(lists[i], lists[i + 1])
                     for i in range(0, len(lists), 2)]
        a, b = lists
        cand = [jnp.maximum(a[i], b[K - 1 - i]) for i in range(K)]
        thresh = cand[0]
        for x in cand[1:]:
            thresh = jnp.minimum(thresh, x)
        rowmax = jnp.maximum(a[0], b[0])
        es = [jnp.where(v >= thresh, jnp.exp(v - rowmax), 0.0) for v in vs]
        inv = 1.0 / _vadd_tree(es)
        for j in range(E):
            buf_out[j, sl] = es[j] * inv


def _sc_route_body(noisy_hbm, out_hbm, buf_in, buf_out):
    rows = buf_in.shape[1]
    wid = lax.axis_index("s") * NC + lax.axis_index("c")
    base = wid * rows
    pltpu.sync_copy(noisy_hbm.at[:, pl.ds(base, rows)], buf_in)
    _route_groups(buf_in, buf_out, rows // LANES)
    pltpu.sync_copy(buf_out, out_hbm.at[:, pl.ds(base, rows)])


def _sc_route(noisy_t):
    t = noisy_t.shape[1]
    rows = t // NW
    mesh = plsc.VectorSubcoreMesh(core_axis_name="c", subcore_axis_name="s")
    return pl.kernel(
        _sc_route_body,
        out_type=jax.ShapeDtypeStruct((E, t), jnp.float32),
        mesh=mesh,
        scratch_types=[
            pltpu.VMEM((E, rows), jnp.float32),
            pltpu.VMEM((E, rows), jnp.float32),
        ],
    )(noisy_t)


@jax.jit
def kernel(x, noise, Wg, Wnoise):
    t = x.shape[0]
    w = jnp.concatenate([Wg, Wnoise], axis=1)
    noisy_t = _tc_noisy_logits_t(x, noise, w)
    return _sc_route(noisy_t).T
